# baseline (speedup 1.0000x reference)
import jax
import jax.numpy as jnp
from jax import lax
from jax.experimental import pallas as pl
from jax.experimental.pallas import tpu as pltpu
from jax.experimental.pallas import tpu_sc as plsc

B, S = 16384, 50
D = 64
NC, NS = 2, 16
NW = NC * NS
B_PER_W = B // NW
CB = 16
CROWS = CB * S
NBUF = 2
N_CHUNKS = B_PER_W // CB
N_GROUPS = N_CHUNKS // NBUF
assert B_PER_W % (CB * NBUF) == 0


def _gather_body(idx_hbm, table_hbm, out_hbm, idx_v, rows_v, isems, gsems, osems):
    wid = lax.axis_index("s") * NC + lax.axis_index("c")
    bbase = wid * B_PER_W

    def issue_idx(j, b):
        row0 = bbase + j * CB
        for r in range(CB):
            pltpu.async_copy(idx_hbm.at[row0 + r], idx_v.at[b, r], isems[b])

    def wait_idx(b):
        for r in range(CB):
            pltpu.make_async_copy(
                idx_hbm.at[bbase], idx_v.at[b, r], isems[b]).wait()

    def issue_gather(b):
        for r in range(CB):
            pltpu.async_copy(
                table_hbm.at[idx_v.at[b, r]], rows_v.at[b, r], gsems[b])

    def wait_gather(b):
        for r in range(CB):
            pltpu.make_async_copy(
                table_hbm.at[idx_v.at[b, r]], rows_v.at[b, r],
                gsems[b]).wait()

    def issue_out(j, b):
        row0 = bbase + j * CB
        for r in range(CB):
            pltpu.async_copy(
                rows_v.at[b, r], out_hbm.at[row0 + r], osems[b])

    def wait_out(b):
        for r in range(CB):
            pltpu.make_async_copy(
                rows_v.at[b, r], out_hbm.at[bbase], osems[b]).wait()

    def finalize(k, b, last):
        wait_gather(b)
        issue_out(k, b)
        if not last:
            issue_idx(jnp.minimum(k + NBUF, N_CHUNKS - 1), b)

    for b in range(NBUF):
        issue_idx(b, b)
    for b in range(NBUF):
        wait_idx(b)
        issue_gather(b)
        if b > 0:
            finalize(b - 1, b - 1, last=False)

    @pl.loop(1, N_GROUPS)
    def _group(g):
        j0 = g * NBUF
        for b in range(NBUF):
            j = j0 + b
            wait_idx(b)
            wait_out(b)
            issue_gather(b)
            pb = (b - 1) % NBUF
            finalize(j - 1, pb, last=False)

    last_b = (N_CHUNKS - 1) % NBUF
    finalize(N_CHUNKS - 1, last_b, last=True)
    for b in range(NBUF):
        wait_out(b)
    for b in range(NBUF):
        if b != last_b:
            wait_idx(b)


def kernel(word_idx, emb):
    mesh = plsc.VectorSubcoreMesh(core_axis_name="c", subcore_axis_name="s")
    f = pl.kernel(
        _gather_body,
        out_type=jax.ShapeDtypeStruct((B, S, D), jnp.float32),
        mesh=mesh,
        scratch_types=[
            pltpu.VMEM((NBUF, CB, S), jnp.int32),
            pltpu.VMEM((NBUF, CB, S, D), jnp.float32),
            [pltpu.SemaphoreType.DMA] * NBUF,
            [pltpu.SemaphoreType.DMA] * NBUF,
            [pltpu.SemaphoreType.DMA] * NBUF,
        ],
        compiler_params=pltpu.CompilerParams(use_tc_tiling_on_sc=False),
    )
    return f(word_idx, emb)


# speedup vs baseline: 1.0122x; 1.0122x over previous
import jax
import jax.numpy as jnp
from jax import lax
from jax.experimental import pallas as pl
from jax.experimental.pallas import tpu as pltpu
from jax.experimental.pallas import tpu_sc as plsc

B, S = 16384, 50
D = 64
N_ROWS = B * S
L = 128
N_IROWS = N_ROWS // L
NC, NS = 2, 16
NW = NC * NS
R_PER_W = N_IROWS // NW
CB = 5
CROWS = CB * L
NBUF = 2
N_CHUNKS = R_PER_W // CB
N_GROUPS = N_CHUNKS // NBUF
assert R_PER_W % (CB * NBUF) == 0


def _gather_body(idx_hbm, table_hbm, out_hbm, idx_v, rows_v, isems, gsems, osems):
    wid = lax.axis_index("s") * NC + lax.axis_index("c")
    rbase = wid * R_PER_W

    def issue_idx(j, b):
        row0 = rbase + j * CB
        for r in range(CB):
            pltpu.async_copy(idx_hbm.at[row0 + r], idx_v.at[b, r], isems[b])

    def wait_idx(b):
        for r in range(CB):
            pltpu.make_async_copy(
                idx_hbm.at[rbase], idx_v.at[b, r], isems[b]).wait()

    def issue_gather(b):
        for r in range(CB):
            pltpu.async_copy(
                table_hbm.at[idx_v.at[b, r]], rows_v.at[b, r], gsems[b])

    def wait_gather(b):
        for r in range(CB):
            pltpu.make_async_copy(
                table_hbm.at[idx_v.at[b, r]], rows_v.at[b, r],
                gsems[b]).wait()

    def issue_out(j, b):
        row0 = rbase + j * CB
        for r in range(CB):
            pltpu.async_copy(
                rows_v.at[b, r], out_hbm.at[pl.ds((row0 + r) * L, L)],
                osems[b])

    def wait_out(b):
        for r in range(CB):
            pltpu.make_async_copy(
                rows_v.at[b, r], out_hbm.at[pl.ds(rbase * L, L)],
                osems[b]).wait()

    def finalize(k, b, last):
        wait_gather(b)
        issue_out(k, b)
        if not last:
            issue_idx(jnp.minimum(k + NBUF, N_CHUNKS - 1), b)

    for b in range(NBUF):
        issue_idx(b, b)
    for b in range(NBUF):
        wait_idx(b)
        issue_gather(b)
        if b > 0:
            finalize(b - 1, b - 1, last=False)

    @pl.loop(1, N_GROUPS)
    def _group(g):
        j0 = g * NBUF
        for b in range(NBUF):
            j = j0 + b
            wait_idx(b)
            wait_out(b)
            issue_gather(b)
            pb = (b - 1) % NBUF
            finalize(j - 1, pb, last=False)

    last_b = (N_CHUNKS - 1) % NBUF
    finalize(N_CHUNKS - 1, last_b, last=True)
    for b in range(NBUF):
        wait_out(b)
    for b in range(NBUF):
        if b != last_b:
            wait_idx(b)


def kernel(word_idx, emb):
    idx2 = word_idx.reshape(N_IROWS, L)
    mesh = plsc.VectorSubcoreMesh(core_axis_name="c", subcore_axis_name="s")
    f = pl.kernel(
        _gather_body,
        out_type=jax.ShapeDtypeStruct((N_ROWS, D), jnp.float32),
        mesh=mesh,
        scratch_types=[
            pltpu.VMEM((NBUF, CB, L), jnp.int32),
            pltpu.VMEM((NBUF, CB, L, D), jnp.float32),
            [pltpu.SemaphoreType.DMA] * NBUF,
            [pltpu.SemaphoreType.DMA] * NBUF,
            [pltpu.SemaphoreType.DMA] * NBUF,
        ],
        compiler_params=pltpu.CompilerParams(use_tc_tiling_on_sc=False),
    )
    out = f(idx2, emb)
    return out.reshape(B, S, D)
